# baseline (speedup 1.0000x reference)
import jax
import jax.numpy as jnp
from jax import lax
from jax.experimental import pallas as pl
from jax.experimental.pallas import tpu as pltpu
from jax.experimental.pallas import tpu_sc as plsc

N_NODES = 100000
N_EDGES = 6400000
BATCH = 4

NC = 2
NS = 16
NW = NC * NS

N_PAD = 100352
ROWS_PER_TILE = N_PAD // NS

CHUNK = 8192
E_PER_W = 204800
N_CHUNKS = E_PER_W // CHUNK
E_PAD = E_PER_W * NW

_mesh = plsc.VectorSubcoreMesh(core_axis_name="c", subcore_axis_name="s")
_sc_params = pltpu.CompilerParams(use_tc_tiling_on_sc=False)


def _deg_body(dst_hbm, degp_hbm, idx_v, ones_v, fill_v, deg_sh):
    c = lax.axis_index("c")
    s = lax.axis_index("s")
    w = c * NS + s
    rs = pl.ds(s * ROWS_PER_TILE, ROWS_PER_TILE)

    def fill_ones(i, carry):
        ones_v[pl.ds(i * 16, 16)] = jnp.full((16,), 1.0, jnp.float32)
        return carry

    lax.fori_loop(0, CHUNK // 16, fill_ones, 0)

    def fill_half(i, carry):
        fill_v[pl.ds(i * 16, 16)] = jnp.full((16,), 0.5, jnp.float32)
        return carry

    lax.fori_loop(0, ROWS_PER_TILE // 16, fill_half, 0)
    pltpu.sync_copy(fill_v, deg_sh.at[rs])
    plsc.subcore_barrier()

    base = w * E_PER_W

    def body(i, carry):
        off = pl.multiple_of(base + i * CHUNK, CHUNK)
        pltpu.sync_copy(dst_hbm.at[pl.ds(off, CHUNK)], idx_v)
        pltpu.sync_copy(ones_v, deg_sh.at[idx_v], add=True)
        return carry

    lax.fori_loop(0, N_CHUNKS, body, 0)
    plsc.subcore_barrier()
    pltpu.sync_copy(deg_sh.at[rs], degp_hbm.at[c].at[rs])


_deg_call = pl.kernel(
    _deg_body,
    out_type=jax.ShapeDtypeStruct((NC, N_PAD), jnp.float32),
    mesh=_mesh,
    compiler_params=_sc_params,
    scratch_types=[
        pltpu.VMEM((CHUNK,), jnp.int32),
        pltpu.VMEM((CHUNK,), jnp.float32),
        pltpu.VMEM((ROWS_PER_TILE,), jnp.float32),
        pltpu.VMEM_SHARED((N_PAD,), jnp.float32),
    ],
)


def _edge_body(src_hbm, dst_hbm, g0_hbm, g1_hbm, g2_hbm, g3_hbm,
               a0_hbm, a1_hbm, a2_hbm, a3_hbm,
               src_v, dst_v, msg0_v, msg1_v, msg2_v, msg3_v, buf_v,
               ac0_sh, ac1_sh, ac2_sh, ac3_sh,
               isem0, isem1, gsem0, gsem1, gsem2, gsem3,
               ssem0, ssem1, ssem2, ssem3):
    c = lax.axis_index("c")
    s = lax.axis_index("s")
    w = c * NS + s
    rs = pl.ds(s * ROWS_PER_TILE, ROWS_PER_TILE)
    g_hbms = (g0_hbm, g1_hbm, g2_hbm, g3_hbm)
    a_hbms = (a0_hbm, a1_hbm, a2_hbm, a3_hbm)
    a_shs = (ac0_sh, ac1_sh, ac2_sh, ac3_sh)
    msgs = (msg0_v, msg1_v, msg2_v, msg3_v)
    gsems = (gsem0, gsem1, gsem2, gsem3)
    ssems = (ssem0, ssem1, ssem2, ssem3)

    def fill_zero(i, carry):
        buf_v[pl.ds(i * 16, 16)] = jnp.full((16,), 0.0, jnp.float32)
        return carry

    lax.fori_loop(0, ROWS_PER_TILE // 16, fill_zero, 0)
    for b in range(BATCH):
        pltpu.sync_copy(buf_v, a_shs[b].at[rs])
    plsc.subcore_barrier()

    base = w * E_PER_W

    def body(i, carry):
        off = pl.multiple_of(base + i * CHUNK, CHUNK)
        di = pltpu.async_copy(src_hbm.at[pl.ds(off, CHUNK)], src_v, isem0)
        dj = pltpu.async_copy(dst_hbm.at[pl.ds(off, CHUNK)], dst_v, isem1)
        di.wait()
        gds = [pltpu.async_copy(g_hbms[b].at[src_v], msgs[b], gsems[b])
               for b in range(BATCH)]
        dj.wait()
        sds = []
        for b in range(BATCH):
            gds[b].wait()
            sds.append(pltpu.async_copy(msgs[b], a_shs[b].at[dst_v],
                                        ssems[b], add=True))
        for b in range(BATCH):
            sds[b].wait()
        return carry

    lax.fori_loop(0, N_CHUNKS, body, 0)
    plsc.subcore_barrier()
    for b in range(BATCH):
        pltpu.sync_copy(a_shs[b].at[rs], a_hbms[b].at[c].at[rs])


_edge_call = pl.kernel(
    _edge_body,
    out_type=tuple(jax.ShapeDtypeStruct((NC, N_PAD), jnp.float32)
                   for _ in range(BATCH)),
    mesh=_mesh,
    compiler_params=_sc_params,
    scratch_types=(
        [pltpu.VMEM((CHUNK,), jnp.int32)] * 2
        + [pltpu.VMEM((CHUNK,), jnp.float32)] * 4
        + [pltpu.VMEM((ROWS_PER_TILE,), jnp.float32)]
        + [pltpu.VMEM_SHARED((N_PAD,), jnp.float32)] * 4
        + [pltpu.SemaphoreType.DMA] * 10
    ),
)


def _prep1_body(w1_ref, degp_ref, x4_ref, g1_ref, dis_ref):
    deg = degp_ref[0:1, :] + degp_ref[1:2, :]
    dis = lax.rsqrt(deg)
    dis_ref[...] = dis
    g1_ref[...] = x4_ref[...] * (dis * w1_ref[0:1, 0:1])


_prep1_call = pl.pallas_call(
    _prep1_body,
    out_shape=(
        jax.ShapeDtypeStruct((BATCH, N_PAD), jnp.float32),
        jax.ShapeDtypeStruct((1, N_PAD), jnp.float32),
    ),
)


def _prep2_body(w2_ref, b1_ref, at_ref, g1t_ref, dis_ref, g2_ref):
    a = at_ref[:, 0, :] + at_ref[:, 1, :]
    y1 = dis_ref[...] * (a + g1t_ref[...]) + b1_ref[0:1, 0:1]
    h = jnp.maximum(y1, 0.0)
    g2_ref[...] = h * (dis_ref[...] * w2_ref[0:1, 0:1])


_prep2_call = pl.pallas_call(
    _prep2_body,
    out_shape=jax.ShapeDtypeStruct((BATCH, N_PAD), jnp.float32),
)


def _out_body(b2_ref, at_ref, g2t_ref, dis_ref, y_ref):
    a = at_ref[:, 0, :] + at_ref[:, 1, :]
    y_ref[...] = dis_ref[...] * (a + g2t_ref[...]) + b2_ref[0:1, 0:1]


_out_call = pl.pallas_call(
    _out_body,
    out_shape=jax.ShapeDtypeStruct((BATCH, N_PAD), jnp.float32),
)


def kernel(x, edge_index, W1, b1, W2, b2):
    n = x.shape[1]
    e = edge_index.shape[1]
    e32 = edge_index.astype(jnp.int32)
    pad = jnp.full((2, E_PAD - e), n, dtype=jnp.int32)
    e32 = jnp.concatenate([e32, pad], axis=1)
    src = e32[0]
    dst = e32[1]

    x4 = jnp.pad(x[:, :, 0], ((0, 0), (0, N_PAD - n)))
    w1 = W1.reshape(1, 1)
    w2 = W2.reshape(1, 1)
    b1r = b1.reshape(1, 1)
    b2r = b2.reshape(1, 1)

    degp = _deg_call(dst)
    g1t, dis = _prep1_call(w1, degp, x4)
    acc1 = _edge_call(src, dst, g1t[0], g1t[1], g1t[2], g1t[3])
    a1t = jnp.stack(acc1, axis=0)
    g2t = _prep2_call(w2, b1r, a1t, g1t, dis)
    acc2 = _edge_call(src, dst, g2t[0], g2t[1], g2t[2], g2t[3])
    a2t = jnp.stack(acc2, axis=0)
    y2t = _out_call(b2r, a2t, g2t, dis)
    return y2t[:, :n][:, :, None]


# speedup vs baseline: 1.5551x; 1.5551x over previous
import jax
import jax.numpy as jnp
from jax import lax
from jax.experimental import pallas as pl
from jax.experimental.pallas import tpu as pltpu
from jax.experimental.pallas import tpu_sc as plsc

N_NODES = 100000
N_EDGES = 6400000
BATCH = 4

NC = 2
NS = 16
NW = NC * NS

N_PAD = 100352
ROWS_PER_TILE = N_PAD // NS

CHUNK = 6400
E_PER_W = 204800
N_CHUNKS = E_PER_W // CHUNK
E_PAD = E_PER_W * NW

_mesh = plsc.VectorSubcoreMesh(core_axis_name="c", subcore_axis_name="s")
_sc_params = pltpu.CompilerParams(use_tc_tiling_on_sc=False)


def _deg_body(dst_hbm, degp_hbm, idx_v, ones_v, fill_v, deg_sh):
    c = lax.axis_index("c")
    s = lax.axis_index("s")
    w = c * NS + s
    rs = pl.ds(s * ROWS_PER_TILE, ROWS_PER_TILE)

    def fill_ones(i, carry):
        ones_v[pl.ds(i * 16, 16)] = jnp.full((16,), 1.0, jnp.float32)
        return carry

    lax.fori_loop(0, CHUNK // 16, fill_ones, 0)

    def fill_half(i, carry):
        fill_v[pl.ds(i * 16, 16)] = jnp.full((16,), 0.5, jnp.float32)
        return carry

    lax.fori_loop(0, ROWS_PER_TILE // 16, fill_half, 0)
    pltpu.sync_copy(fill_v, deg_sh.at[rs])
    plsc.subcore_barrier()

    base = w * E_PER_W

    def body(i, carry):
        off = pl.multiple_of(base + i * CHUNK, CHUNK)
        pltpu.sync_copy(dst_hbm.at[pl.ds(off, CHUNK)], idx_v)
        pltpu.sync_copy(ones_v, deg_sh.at[idx_v], add=True)
        return carry

    lax.fori_loop(0, N_CHUNKS, body, 0)
    plsc.subcore_barrier()
    pltpu.sync_copy(deg_sh.at[rs], degp_hbm.at[c].at[rs])


_deg_call = pl.kernel(
    _deg_body,
    out_type=jax.ShapeDtypeStruct((NC, N_PAD), jnp.float32),
    mesh=_mesh,
    compiler_params=_sc_params,
    scratch_types=[
        pltpu.VMEM((CHUNK,), jnp.int32),
        pltpu.VMEM((CHUNK,), jnp.float32),
        pltpu.VMEM((ROWS_PER_TILE,), jnp.float32),
        pltpu.VMEM_SHARED((N_PAD,), jnp.float32),
    ],
)


def _edge_body(src_hbm, dst_hbm, g0_hbm, g1_hbm, g2_hbm, g3_hbm,
               a0_hbm, a1_hbm, a2_hbm, a3_hbm,
               src0_v, dst0_v, src1_v, dst1_v,
               m00, m01, m02, m03, m10, m11, m12, m13,
               g0_sh, g1_sh, g2_sh, g3_sh, ac0_sh, ac1_sh, ac2_sh, ac3_sh,
               is0, id0, is1, id1,
               gsem0, gsem1, gsem2, gsem3,
               ss00, ss01, ss02, ss03, ss10, ss11, ss12, ss13):
    c = lax.axis_index("c")
    s = lax.axis_index("s")
    w = c * NS + s
    rs = pl.ds(s * ROWS_PER_TILE, ROWS_PER_TILE)
    g_hbms = (g0_hbm, g1_hbm, g2_hbm, g3_hbm)
    a_hbms = (a0_hbm, a1_hbm, a2_hbm, a3_hbm)
    g_shs = (g0_sh, g1_sh, g2_sh, g3_sh)
    a_shs = (ac0_sh, ac1_sh, ac2_sh, ac3_sh)
    srcs = (src0_v, src1_v)
    dsts = (dst0_v, dst1_v)
    isems = ((is0, id0), (is1, id1))
    msgs = ((m00, m01, m02, m03), (m10, m11, m12, m13))
    gsems = (gsem0, gsem1, gsem2, gsem3)
    ssems = ((ss00, ss01, ss02, ss03), (ss10, ss11, ss12, ss13))
    base = w * E_PER_W

    stage = m00.at[pl.ds(0, ROWS_PER_TILE)]
    for b in range(BATCH):
        pltpu.sync_copy(g_hbms[b].at[rs], stage)
        pltpu.sync_copy(stage, g_shs[b].at[rs])

    def fill_zero(i, carry):
        m00[pl.ds(i * 16, 16)] = jnp.full((16,), 0.0, jnp.float32)
        return carry

    lax.fori_loop(0, ROWS_PER_TILE // 16, fill_zero, 0)
    for b in range(BATCH):
        pltpu.sync_copy(stage, a_shs[b].at[rs])
    plsc.subcore_barrier()

    def load_idx(j, p):
        off = pl.multiple_of(base + j * CHUNK, CHUNK)
        pltpu.async_copy(src_hbm.at[pl.ds(off, CHUNK)], srcs[p], isems[p][0])
        pltpu.async_copy(dst_hbm.at[pl.ds(off, CHUNK)], dsts[p], isems[p][1])

    def wait_idx(p):
        pltpu.make_async_copy(src_hbm.at[pl.ds(0, CHUNK)], srcs[p],
                              isems[p][0]).wait()
        pltpu.make_async_copy(dst_hbm.at[pl.ds(0, CHUNK)], dsts[p],
                              isems[p][1]).wait()

    def start_gathers(p):
        for b in range(BATCH):
            pltpu.async_copy(g_shs[b].at[srcs[p]], msgs[p][b], gsems[b])

    def wait_gathers(p):
        for b in range(BATCH):
            pltpu.make_async_copy(g_shs[b].at[srcs[p]], msgs[p][b],
                                  gsems[b]).wait()

    def start_scats(p):
        for b in range(BATCH):
            pltpu.async_copy(msgs[p][b], a_shs[b].at[dsts[p]],
                             ssems[p][b], add=True)

    def wait_scats(p):
        for b in range(BATCH):
            pltpu.make_async_copy(msgs[p][b], a_shs[b].at[dsts[p]],
                                  ssems[p][b]).wait()

    load_idx(0, 0)
    wait_idx(0)
    start_gathers(0)
    load_idx(1, 1)
    wait_gathers(0)
    start_scats(0)

    def body(i, carry):
        pred = lax.rem(i, 2) == 0

        def one(p, q):
            wait_idx(p)
            start_gathers(p)
            wait_scats(q)
            @pl.when(i + 1 < N_CHUNKS)
            def _():
                load_idx(i + 1, q)
            wait_gathers(p)
            start_scats(p)

        @pl.when(pred)
        def _():
            one(0, 1)

        @pl.when(jnp.logical_not(pred))
        def _():
            one(1, 0)

        return carry

    lax.fori_loop(1, N_CHUNKS, body, 0)
    wait_scats((N_CHUNKS - 1) % 2)
    plsc.subcore_barrier()
    for b in range(BATCH):
        pltpu.sync_copy(a_shs[b].at[rs], a_hbms[b].at[c].at[rs])


_edge_call = pl.kernel(
    _edge_body,
    out_type=tuple(jax.ShapeDtypeStruct((NC, N_PAD), jnp.float32)
                   for _ in range(BATCH)),
    mesh=_mesh,
    compiler_params=_sc_params,
    scratch_types=(
        [pltpu.VMEM((CHUNK,), jnp.int32)] * 4
        + [pltpu.VMEM((CHUNK,), jnp.float32)] * 8
        + [pltpu.VMEM_SHARED((N_PAD,), jnp.float32)] * 8
        + [pltpu.SemaphoreType.DMA] * 16
    ),
)


def _prep1_body(w1_ref, degp_ref, x4_ref, g1_ref, dis_ref):
    deg = degp_ref[0:1, :] + degp_ref[1:2, :]
    dis = lax.rsqrt(deg)
    dis_ref[...] = dis
    g1_ref[...] = x4_ref[...] * (dis * w1_ref[0:1, 0:1])


_prep1_call = pl.pallas_call(
    _prep1_body,
    out_shape=(
        jax.ShapeDtypeStruct((BATCH, N_PAD), jnp.float32),
        jax.ShapeDtypeStruct((1, N_PAD), jnp.float32),
    ),
)


def _prep2_body(w2_ref, b1_ref, at_ref, g1t_ref, dis_ref, g2_ref):
    a = at_ref[:, 0, :] + at_ref[:, 1, :]
    y1 = dis_ref[...] * (a + g1t_ref[...]) + b1_ref[0:1, 0:1]
    h = jnp.maximum(y1, 0.0)
    g2_ref[...] = h * (dis_ref[...] * w2_ref[0:1, 0:1])


_prep2_call = pl.pallas_call(
    _prep2_body,
    out_shape=jax.ShapeDtypeStruct((BATCH, N_PAD), jnp.float32),
)


def _out_body(b2_ref, at_ref, g2t_ref, dis_ref, y_ref):
    a = at_ref[:, 0, :] + at_ref[:, 1, :]
    y_ref[...] = dis_ref[...] * (a + g2t_ref[...]) + b2_ref[0:1, 0:1]


_out_call = pl.pallas_call(
    _out_body,
    out_shape=jax.ShapeDtypeStruct((BATCH, N_PAD), jnp.float32),
)


def kernel(x, edge_index, W1, b1, W2, b2):
    n = x.shape[1]
    e = edge_index.shape[1]
    e32 = edge_index.astype(jnp.int32)
    pad = jnp.full((2, E_PAD - e), n, dtype=jnp.int32)
    e32 = jnp.concatenate([e32, pad], axis=1)
    src = e32[0]
    dst = e32[1]

    x4 = jnp.pad(x[:, :, 0], ((0, 0), (0, N_PAD - n)))
    w1 = W1.reshape(1, 1)
    w2 = W2.reshape(1, 1)
    b1r = b1.reshape(1, 1)
    b2r = b2.reshape(1, 1)

    degp = _deg_call(dst)
    g1t, dis = _prep1_call(w1, degp, x4)
    acc1 = _edge_call(src, dst, g1t[0], g1t[1], g1t[2], g1t[3])
    a1t = jnp.stack(acc1, axis=0)
    g2t = _prep2_call(w2, b1r, a1t, g1t, dis)
    acc2 = _edge_call(src, dst, g2t[0], g2t[1], g2t[2], g2t[3])
    a2t = jnp.stack(acc2, axis=0)
    y2t = _out_call(b2r, a2t, g2t, dis)
    return y2t[:, :n][:, :, None]


# speedup vs baseline: 2.2793x; 1.4656x over previous
import jax
import jax.numpy as jnp
from jax import lax
from jax.experimental import pallas as pl
from jax.experimental.pallas import tpu as pltpu
from jax.experimental.pallas import tpu_sc as plsc

N_NODES = 100000
N_EDGES = 6400000
BATCH = 4

NC = 2
NS = 16
NW = NC * NS

N_PAD = 100352
ROWS_PER_TILE = N_PAD // NS

CHUNK = 5000
E_PER_W = N_EDGES // NW
N_CHUNKS = E_PER_W // CHUNK

_mesh = plsc.VectorSubcoreMesh(core_axis_name="c", subcore_axis_name="s")
_sc_params = pltpu.CompilerParams(use_tc_tiling_on_sc=False)
_sc_params_nl = pltpu.CompilerParams(use_tc_tiling_on_sc=False,
                                     needs_layout_passes=False)


def _deg_body(dst_hbm, degp_hbm, idx_v, ones_v, fill_v, deg_sh):
    c = lax.axis_index("c")
    s = lax.axis_index("s")
    w = c * NS + s
    rs = pl.ds(s * ROWS_PER_TILE, ROWS_PER_TILE)

    def fill_ones(i, carry):
        ones_v[pl.ds(i * 16, 16)] = jnp.full((16,), 1.0, jnp.float32)
        return carry

    lax.fori_loop(0, CHUNK // 16, fill_ones, 0)

    def fill_half(i, carry):
        fill_v[pl.ds(i * 16, 16)] = jnp.full((16,), 0.5, jnp.float32)
        return carry

    lax.fori_loop(0, ROWS_PER_TILE // 16, fill_half, 0)
    pltpu.sync_copy(fill_v, deg_sh.at[rs])
    plsc.subcore_barrier()

    base = w * E_PER_W

    def body(i, carry):
        off = pl.multiple_of(base + i * CHUNK, CHUNK)
        pltpu.sync_copy(dst_hbm.at[pl.ds(off, CHUNK)], idx_v)
        pltpu.sync_copy(ones_v, deg_sh.at[idx_v], add=True)
        return carry

    lax.fori_loop(0, N_CHUNKS, body, 0)
    plsc.subcore_barrier()
    pltpu.sync_copy(deg_sh.at[rs], degp_hbm.at[c].at[rs])


_deg_call = pl.kernel(
    _deg_body,
    out_type=jax.ShapeDtypeStruct((NC, N_PAD), jnp.float32),
    mesh=_mesh,
    compiler_params=_sc_params,
    scratch_types=[
        pltpu.VMEM((CHUNK,), jnp.int32),
        pltpu.VMEM((CHUNK,), jnp.float32),
        pltpu.VMEM((ROWS_PER_TILE,), jnp.float32),
        pltpu.VMEM_SHARED((N_PAD,), jnp.float32),
    ],
)


def _edge_body(src_hbm, dst_hbm, p01_hbm, p23_hbm,
               a0_hbm, a1_hbm, a2_hbm, a3_hbm,
               src0_v, dst0_v, src1_v, dst1_v,
               pk0, pk1,
               m00, m01, m02, m03, m10, m11, m12, m13,
               stf_v, sti_v,
               p01_sh, p23_sh, ac0_sh, ac1_sh, ac2_sh, ac3_sh,
               is0, id0, is1, id1, gsem0, gsem1,
               ss00, ss01, ss02, ss03, ss10, ss11, ss12, ss13):
    c = lax.axis_index("c")
    s = lax.axis_index("s")
    w = c * NS + s
    rs = pl.ds(s * ROWS_PER_TILE, ROWS_PER_TILE)
    p_hbms = (p01_hbm, p23_hbm)
    a_hbms = (a0_hbm, a1_hbm, a2_hbm, a3_hbm)
    p_shs = (p01_sh, p23_sh)
    a_shs = (ac0_sh, ac1_sh, ac2_sh, ac3_sh)
    srcs = (src0_v, src1_v)
    dsts = (dst0_v, dst1_v)
    isems = ((is0, id0), (is1, id1))
    pks = (pk0, pk1)
    msgs = ((m00, m01, m02, m03), (m10, m11, m12, m13))
    gsems = (gsem0, gsem1)
    ssems = ((ss00, ss01, ss02, ss03), (ss10, ss11, ss12, ss13))
    base = w * E_PER_W

    for t in range(2):
        pltpu.sync_copy(p_hbms[t].at[rs], sti_v)
        pltpu.sync_copy(sti_v, p_shs[t].at[rs])

    def fill_zero(i, carry):
        stf_v[pl.ds(i * 16, 16)] = jnp.full((16,), 0.0, jnp.float32)
        return carry

    lax.fori_loop(0, ROWS_PER_TILE // 16, fill_zero, 0)
    for b in range(BATCH):
        pltpu.sync_copy(stf_v, a_shs[b].at[rs])
    plsc.subcore_barrier()

    def load_idx(j, p):
        off = pl.multiple_of(base + j * CHUNK, 8)
        pltpu.async_copy(src_hbm.at[pl.ds(off, CHUNK)], srcs[p], isems[p][0])
        pltpu.async_copy(dst_hbm.at[pl.ds(off, CHUNK)], dsts[p], isems[p][1])

    def wait_idx(p):
        pltpu.make_async_copy(src_hbm.at[pl.ds(0, CHUNK)], srcs[p],
                              isems[p][0]).wait()
        pltpu.make_async_copy(dst_hbm.at[pl.ds(0, CHUNK)], dsts[p],
                              isems[p][1]).wait()

    def start_gathers(p):
        for t in range(2):
            pltpu.async_copy(p_shs[t].at[srcs[p]], pks[t], gsems[t])

    def wait_gathers(p):
        for t in range(2):
            pltpu.make_async_copy(p_shs[t].at[srcs[p]], pks[t],
                                  gsems[t]).wait()

    def unpack_chunk(p):
        for t in range(2):
            ua = msgs[p][2 * t]
            ub = msgs[p][2 * t + 1]

            def u(k, carry, _t=t, _ua=ua, _ub=ub):
                sl = pl.ds(k * 16, 16)
                ab = plsc.bitcast(pks[_t][sl], jnp.bfloat16)
                a, b = plsc.unpack(ab, format=plsc.PackFormat.INTERLEAVED)
                _ua[sl] = a
                _ub[sl] = b
                return carry

            lax.fori_loop(0, CHUNK // 16, u, 0)
            sl = pl.ds(CHUNK - 16, 16)
            ab = plsc.bitcast(pks[t][sl], jnp.bfloat16)
            a, b = plsc.unpack(ab, format=plsc.PackFormat.INTERLEAVED)
            ua[sl] = a
            ub[sl] = b

    def start_scats(p):
        for b in range(BATCH):
            pltpu.async_copy(msgs[p][b], a_shs[b].at[dsts[p]],
                             ssems[p][b], add=True)

    def wait_scats(p):
        for b in range(BATCH):
            pltpu.make_async_copy(msgs[p][b], a_shs[b].at[dsts[p]],
                                  ssems[p][b]).wait()

    load_idx(0, 0)
    wait_idx(0)
    start_gathers(0)
    load_idx(1, 1)
    wait_gathers(0)
    unpack_chunk(0)
    start_scats(0)

    def body(i, carry):
        pred = lax.rem(i, 2) == 0

        def one(p, q):
            wait_idx(p)
            start_gathers(p)
            wait_scats(q)
            @pl.when(i + 1 < N_CHUNKS)
            def _():
                load_idx(i + 1, q)
            wait_gathers(p)
            unpack_chunk(p)
            start_scats(p)

        @pl.when(pred)
        def _():
            one(0, 1)

        @pl.when(jnp.logical_not(pred))
        def _():
            one(1, 0)

        return carry

    lax.fori_loop(1, N_CHUNKS, body, 0)
    wait_scats((N_CHUNKS - 1) % 2)
    plsc.subcore_barrier()
    for b in range(BATCH):
        pltpu.sync_copy(a_shs[b].at[rs], a_hbms[b].at[c].at[rs])


_edge_call = pl.kernel(
    _edge_body,
    out_type=tuple(jax.ShapeDtypeStruct((NC, N_PAD), jnp.float32)
                   for _ in range(BATCH)),
    mesh=_mesh,
    compiler_params=_sc_params_nl,
    scratch_types=(
        [pltpu.VMEM((CHUNK,), jnp.int32)] * 4
        + [pltpu.VMEM((CHUNK,), jnp.int32)] * 2
        + [pltpu.VMEM((CHUNK,), jnp.float32)] * 8
        + [pltpu.VMEM((ROWS_PER_TILE,), jnp.float32)]
        + [pltpu.VMEM((ROWS_PER_TILE,), jnp.int32)]
        + [pltpu.VMEM_SHARED((N_PAD,), jnp.int32)] * 2
        + [pltpu.VMEM_SHARED((N_PAD,), jnp.float32)] * 4
        + [pltpu.SemaphoreType.DMA] * 14
    ),
)


def _prep1_body(w1_ref, degp_ref, x4_ref, g1_ref, dis_ref):
    deg = degp_ref[0:1, :] + degp_ref[1:2, :]
    dis = lax.rsqrt(deg)
    dis_ref[...] = dis
    g1_ref[...] = x4_ref[...] * (dis * w1_ref[0:1, 0:1])


_prep1_call = pl.pallas_call(
    _prep1_body,
    out_shape=(
        jax.ShapeDtypeStruct((BATCH, N_PAD), jnp.float32),
        jax.ShapeDtypeStruct((1, N_PAD), jnp.float32),
    ),
)


def _prep2_body(w2_ref, b1_ref, at_ref, g1t_ref, dis_ref, g2_ref):
    a = at_ref[:, 0, :] + at_ref[:, 1, :]
    y1 = dis_ref[...] * (a + g1t_ref[...]) + b1_ref[0:1, 0:1]
    h = jnp.maximum(y1, 0.0)
    g2_ref[...] = h * (dis_ref[...] * w2_ref[0:1, 0:1])


_prep2_call = pl.pallas_call(
    _prep2_body,
    out_shape=jax.ShapeDtypeStruct((BATCH, N_PAD), jnp.float32),
)


def _out_body(b2_ref, at_ref, g2t_ref, dis_ref, y_ref):
    a = at_ref[:, 0, :] + at_ref[:, 1, :]
    y_ref[...] = dis_ref[...] * (a + g2t_ref[...]) + b2_ref[0:1, 0:1]


_out_call = pl.pallas_call(
    _out_body,
    out_shape=jax.ShapeDtypeStruct((BATCH, N_PAD), jnp.float32),
)


def _pack2(a, b):
    ab = jnp.stack([a, b], axis=-1).astype(jnp.bfloat16)
    return jax.lax.bitcast_convert_type(ab, jnp.int32)


def kernel(x, edge_index, W1, b1, W2, b2):
    n = x.shape[1]
    e32 = edge_index.astype(jnp.int32)
    src = e32[0]
    dst = e32[1]

    x4 = jnp.pad(x[:, :, 0], ((0, 0), (0, N_PAD - n)))
    w1 = W1.reshape(1, 1)
    w2 = W2.reshape(1, 1)
    b1r = b1.reshape(1, 1)
    b2r = b2.reshape(1, 1)

    degp = _deg_call(dst)
    g1t, dis = _prep1_call(w1, degp, x4)
    acc1 = _edge_call(src, dst, _pack2(g1t[0], g1t[1]),
                      _pack2(g1t[2], g1t[3]))
    a1t = jnp.stack(acc1, axis=0)
    g2t = _prep2_call(w2, b1r, a1t, g1t, dis)
    acc2 = _edge_call(src, dst, _pack2(g2t[0], g2t[1]),
                      _pack2(g2t[2], g2t[3]))
    a2t = jnp.stack(acc2, axis=0)
    y2t = _out_call(b2r, a2t, g2t, dis)
    return y2t[:, :n][:, :, None]


# speedup vs baseline: 2.2822x; 1.0013x over previous
import jax
import jax.numpy as jnp
from jax import lax
from jax.experimental import pallas as pl
from jax.experimental.pallas import tpu as pltpu
from jax.experimental.pallas import tpu_sc as plsc

N_NODES = 100000
N_EDGES = 6400000
BATCH = 4

NC = 2
NS = 16
NW = NC * NS

N_PAD = 100352
ROWS_PER_TILE = N_PAD // NS

CHUNK = 5000
E_PER_W = N_EDGES // NW
N_CHUNKS = E_PER_W // CHUNK

_mesh = plsc.VectorSubcoreMesh(core_axis_name="c", subcore_axis_name="s")
_sc_params = pltpu.CompilerParams(use_tc_tiling_on_sc=False)
_sc_params_nl = pltpu.CompilerParams(use_tc_tiling_on_sc=False,
                                     needs_layout_passes=False)


def _deg_body(dst_hbm, degp_hbm, idx_v, ones_v, fill_v, deg_sh):
    c = lax.axis_index("c")
    s = lax.axis_index("s")
    w = c * NS + s
    rs = pl.ds(s * ROWS_PER_TILE, ROWS_PER_TILE)

    def fill_ones(i, carry):
        ones_v[pl.ds(i * 16, 16)] = jnp.full((16,), 1.0, jnp.float32)
        return carry

    lax.fori_loop(0, CHUNK // 16, fill_ones, 0)

    def fill_half(i, carry):
        fill_v[pl.ds(i * 16, 16)] = jnp.full((16,), 0.5, jnp.float32)
        return carry

    lax.fori_loop(0, ROWS_PER_TILE // 16, fill_half, 0)
    pltpu.sync_copy(fill_v, deg_sh.at[rs])
    plsc.subcore_barrier()

    base = w * E_PER_W

    def body(i, carry):
        off = pl.multiple_of(base + i * CHUNK, CHUNK)
        pltpu.sync_copy(dst_hbm.at[pl.ds(off, CHUNK)], idx_v)
        pltpu.sync_copy(ones_v, deg_sh.at[idx_v], add=True)
        return carry

    lax.fori_loop(0, N_CHUNKS, body, 0)
    plsc.subcore_barrier()
    pltpu.sync_copy(deg_sh.at[rs], degp_hbm.at[c].at[rs])


_deg_call = pl.kernel(
    _deg_body,
    out_type=jax.ShapeDtypeStruct((NC, N_PAD), jnp.float32),
    mesh=_mesh,
    compiler_params=_sc_params,
    scratch_types=[
        pltpu.VMEM((CHUNK,), jnp.int32),
        pltpu.VMEM((CHUNK,), jnp.float32),
        pltpu.VMEM((ROWS_PER_TILE,), jnp.float32),
        pltpu.VMEM_SHARED((N_PAD,), jnp.float32),
    ],
)


def _edge_body(src_hbm, dst_hbm, p01_hbm, p23_hbm,
               a0_hbm, a1_hbm, a2_hbm, a3_hbm,
               src0_v, dst0_v, src1_v, dst1_v,
               pk0, pk1,
               m00, m01, m02, m03, m10, m11, m12, m13,
               stf_v, sti_v,
               p01_sh, p23_sh, ac0_sh, ac1_sh, ac2_sh, ac3_sh,
               is0, id0, is1, id1, gsem0, gsem1,
               ss00, ss01, ss02, ss03, ss10, ss11, ss12, ss13):
    c = lax.axis_index("c")
    s = lax.axis_index("s")
    w = c * NS + s
    rs = pl.ds(s * ROWS_PER_TILE, ROWS_PER_TILE)
    p_hbms = (p01_hbm, p23_hbm)
    a_hbms = (a0_hbm, a1_hbm, a2_hbm, a3_hbm)
    p_shs = (p01_sh, p23_sh)
    a_shs = (ac0_sh, ac1_sh, ac2_sh, ac3_sh)
    srcs = (src0_v, src1_v)
    dsts = (dst0_v, dst1_v)
    isems = ((is0, id0), (is1, id1))
    pks = (pk0, pk1)
    msgs = ((m00, m01, m02, m03), (m10, m11, m12, m13))
    gsems = (gsem0, gsem1)
    ssems = ((ss00, ss01, ss02, ss03), (ss10, ss11, ss12, ss13))
    base = w * E_PER_W

    for t in range(2):
        pltpu.sync_copy(p_hbms[t].at[rs], sti_v)
        pltpu.sync_copy(sti_v, p_shs[t].at[rs])

    def fill_zero(i, carry):
        stf_v[pl.ds(i * 16, 16)] = jnp.full((16,), 0.0, jnp.float32)
        return carry

    lax.fori_loop(0, ROWS_PER_TILE // 16, fill_zero, 0)
    for b in range(BATCH):
        pltpu.sync_copy(stf_v, a_shs[b].at[rs])
    plsc.subcore_barrier()

    def load_idx(j, p):
        off = pl.multiple_of(base + j * CHUNK, 8)
        pltpu.async_copy(src_hbm.at[pl.ds(off, CHUNK)], srcs[p], isems[p][0])
        pltpu.async_copy(dst_hbm.at[pl.ds(off, CHUNK)], dsts[p], isems[p][1])

    def wait_idx(p):
        pltpu.make_async_copy(src_hbm.at[pl.ds(0, CHUNK)], srcs[p],
                              isems[p][0]).wait()
        pltpu.make_async_copy(dst_hbm.at[pl.ds(0, CHUNK)], dsts[p],
                              isems[p][1]).wait()

    def start_gathers(p):
        for t in range(2):
            pltpu.async_copy(p_shs[t].at[srcs[p]], pks[t], gsems[t])

    def wait_gathers(p):
        for t in range(2):
            pltpu.make_async_copy(p_shs[t].at[srcs[p]], pks[t],
                                  gsems[t]).wait()

    def unpack_chunk(p):
        for t in range(2):
            ua = msgs[p][2 * t]
            ub = msgs[p][2 * t + 1]

            def u(k, carry, _t=t, _ua=ua, _ub=ub):
                sl = pl.ds(k * 16, 16)
                ab = plsc.bitcast(pks[_t][sl], jnp.bfloat16)
                a, b = plsc.unpack(ab, format=plsc.PackFormat.INTERLEAVED)
                _ua[sl] = a
                _ub[sl] = b
                return carry

            lax.fori_loop(0, CHUNK // 16, u, 0)
            sl = pl.ds(CHUNK - 16, 16)
            ab = plsc.bitcast(pks[t][sl], jnp.bfloat16)
            a, b = plsc.unpack(ab, format=plsc.PackFormat.INTERLEAVED)
            ua[sl] = a
            ub[sl] = b

    def start_scats(p):
        for b in range(BATCH):
            pltpu.async_copy(msgs[p][b], a_shs[b].at[dsts[p]],
                             ssems[p][b], add=True)

    def wait_scats(p):
        for b in range(BATCH):
            pltpu.make_async_copy(msgs[p][b], a_shs[b].at[dsts[p]],
                                  ssems[p][b]).wait()

    load_idx(0, 0)
    wait_idx(0)
    start_gathers(0)
    load_idx(1, 1)
    wait_gathers(0)
    unpack_chunk(0)
    start_scats(0)

    def body(i, carry):
        pred = lax.rem(i, 2) == 0

        def one(p, q):
            wait_idx(p)
            start_gathers(p)
            wait_scats(q)
            @pl.when(i + 1 < N_CHUNKS)
            def _():
                load_idx(i + 1, q)
            wait_gathers(p)
            unpack_chunk(p)
            start_scats(p)

        @pl.when(pred)
        def _():
            one(0, 1)

        @pl.when(jnp.logical_not(pred))
        def _():
            one(1, 0)

        return carry

    lax.fori_loop(1, N_CHUNKS, body, 0)
    wait_scats((N_CHUNKS - 1) % 2)
    plsc.subcore_barrier()
    for b in range(BATCH):
        pltpu.sync_copy(a_shs[b].at[rs], a_hbms[b].at[c].at[rs])


_edge_call = pl.kernel(
    _edge_body,
    out_type=tuple(jax.ShapeDtypeStruct((NC, N_PAD), jnp.float32)
                   for _ in range(BATCH)),
    mesh=_mesh,
    compiler_params=_sc_params_nl,
    scratch_types=(
        [pltpu.VMEM((CHUNK,), jnp.int32)] * 4
        + [pltpu.VMEM((CHUNK,), jnp.int32)] * 2
        + [pltpu.VMEM((CHUNK,), jnp.float32)] * 8
        + [pltpu.VMEM((ROWS_PER_TILE,), jnp.float32)]
        + [pltpu.VMEM((ROWS_PER_TILE,), jnp.int32)]
        + [pltpu.VMEM_SHARED((N_PAD,), jnp.int32)] * 2
        + [pltpu.VMEM_SHARED((N_PAD,), jnp.float32)] * 4
        + [pltpu.SemaphoreType.DMA] * 14
    ),
)


def _packrows(g):
    lo = jnp.concatenate([g[0:1, :], g[2:3, :]], axis=0)
    hi = jnp.concatenate([g[1:2, :], g[3:4, :]], axis=0)
    lo = jax.lax.bitcast_convert_type(
        lo.astype(jnp.bfloat16), jnp.uint16).astype(jnp.int32)
    hi = jax.lax.bitcast_convert_type(
        hi.astype(jnp.bfloat16), jnp.uint16).astype(jnp.int32)
    return lo | (hi << 16)


def _prep1_body(w1_ref, degp_ref, x4_ref, g1_ref, dis_ref, p1_ref):
    deg = degp_ref[0:1, :] + degp_ref[1:2, :]
    dis = lax.rsqrt(deg)
    dis_ref[...] = dis
    g1 = x4_ref[...] * (dis * w1_ref[0:1, 0:1])
    g1_ref[...] = g1
    p1_ref[...] = _packrows(g1)


_prep1_call = pl.pallas_call(
    _prep1_body,
    out_shape=(
        jax.ShapeDtypeStruct((BATCH, N_PAD), jnp.float32),
        jax.ShapeDtypeStruct((1, N_PAD), jnp.float32),
        jax.ShapeDtypeStruct((2, N_PAD), jnp.int32),
    ),
)


def _prep2_body(w2_ref, b1_ref, a0_ref, a1_ref, a2_ref, a3_ref,
                g1t_ref, dis_ref, g2_ref, p2_ref):
    a = jnp.concatenate([
        a0_ref[0:1, :] + a0_ref[1:2, :],
        a1_ref[0:1, :] + a1_ref[1:2, :],
        a2_ref[0:1, :] + a2_ref[1:2, :],
        a3_ref[0:1, :] + a3_ref[1:2, :],
    ], axis=0)
    y1 = dis_ref[...] * (a + g1t_ref[...]) + b1_ref[0:1, 0:1]
    h = jnp.maximum(y1, 0.0)
    g2 = h * (dis_ref[...] * w2_ref[0:1, 0:1])
    g2_ref[...] = g2
    p2_ref[...] = _packrows(g2)


_prep2_call = pl.pallas_call(
    _prep2_body,
    out_shape=(
        jax.ShapeDtypeStruct((BATCH, N_PAD), jnp.float32),
        jax.ShapeDtypeStruct((2, N_PAD), jnp.int32),
    ),
)


def _out_body(b2_ref, a0_ref, a1_ref, a2_ref, a3_ref, g2t_ref, dis_ref,
              y_ref):
    a = jnp.concatenate([
        a0_ref[0:1, :] + a0_ref[1:2, :],
        a1_ref[0:1, :] + a1_ref[1:2, :],
        a2_ref[0:1, :] + a2_ref[1:2, :],
        a3_ref[0:1, :] + a3_ref[1:2, :],
    ], axis=0)
    y_ref[...] = dis_ref[...] * (a + g2t_ref[...]) + b2_ref[0:1, 0:1]


_out_call = pl.pallas_call(
    _out_body,
    out_shape=jax.ShapeDtypeStruct((BATCH, N_PAD), jnp.float32),
)


def kernel(x, edge_index, W1, b1, W2, b2):
    n = x.shape[1]
    e32 = edge_index.astype(jnp.int32)
    src = e32[0]
    dst = e32[1]

    x4 = jnp.pad(x[:, :, 0], ((0, 0), (0, N_PAD - n)))
    w1 = W1.reshape(1, 1)
    w2 = W2.reshape(1, 1)
    b1r = b1.reshape(1, 1)
    b2r = b2.reshape(1, 1)

    degp = _deg_call(dst)
    g1t, dis, p1 = _prep1_call(w1, degp, x4)
    acc1 = _edge_call(src, dst, p1[0], p1[1])
    g2t, p2 = _prep2_call(w2, b1r, acc1[0], acc1[1], acc1[2], acc1[3],
                          g1t, dis)
    acc2 = _edge_call(src, dst, p2[0], p2[1])
    y2t = _out_call(b2r, acc2[0], acc2[1], acc2[2], acc2[3], g2t, dis)
    return y2t[:, :n][:, :, None]
